# baseline (speedup 1.0000x reference)
import functools

import jax
import jax.numpy as jnp
from jax import lax
from jax.experimental import pallas as pl
from jax.experimental.pallas import tpu as pltpu
from jax.experimental.pallas import tpu_sc as plsc

NU = 100000
NI = 10000
EDG = 1600000
DIN = 128
H = 32
LBL = 400000

K = 128
EP = 1605632
EP2 = EP + 2 * K
LP = 401408
LP2 = LP + 2 * K
NLOC_U = 100096
NLOC_I = 12288
HALF_U = 50000


def _mm2_body(x_ref, wl_ref, bl_ref, wr_ref, br_ref, ol_ref, or_ref):
    x = x_ref[...]
    ol_ref[...] = jnp.dot(x, wl_ref[...], preferred_element_type=jnp.float32) + bl_ref[...]
    or_ref[...] = jnp.dot(x, wr_ref[...], preferred_element_type=jnp.float32) + br_ref[...]


def _mm2(x, wl, bl, wr, br, bn=2048):
    n, din = x.shape
    dout = wl.shape[1]
    return pl.pallas_call(
        _mm2_body,
        grid=(n // bn,),
        in_specs=[
            pl.BlockSpec((bn, din), lambda i: (i, 0)),
            pl.BlockSpec((din, dout), lambda i: (0, 0)),
            pl.BlockSpec((1, dout), lambda i: (0, 0)),
            pl.BlockSpec((din, dout), lambda i: (0, 0)),
            pl.BlockSpec((1, dout), lambda i: (0, 0)),
        ],
        out_specs=[
            pl.BlockSpec((bn, dout), lambda i: (i, 0)),
            pl.BlockSpec((bn, dout), lambda i: (i, 0)),
        ],
        out_shape=[
            jax.ShapeDtypeStruct((n, dout), jnp.float32),
            jax.ShapeDtypeStruct((n, dout), jnp.float32),
        ],
    )(x, wl, bl.reshape(1, -1), wr, br.reshape(1, -1))


def _cmm_body(num_ref, den_ref, bias_ref, wl_ref, bl_ref, wr_ref, br_ref,
              ol_ref, or_ref, *, relu):
    num = jnp.sum(num_ref[...], axis=0)
    den = jnp.sum(den_ref[...], axis=1, keepdims=True) + 1e-30
    z = num / den + bias_ref[...]
    if relu:
        z = jnp.maximum(z, 0.0)
    ol_ref[...] = jnp.dot(z, wl_ref[...], preferred_element_type=jnp.float32) + bl_ref[...]
    or_ref[...] = jnp.dot(z, wr_ref[...], preferred_element_type=jnp.float32) + br_ref[...]


def _cmm(num, den, bias, wl, bl, wr, br, relu, bn=2048):
    c, n, h = num.shape
    cd = den.shape[1]
    dout = wl.shape[1]
    return pl.pallas_call(
        functools.partial(_cmm_body, relu=relu),
        grid=(n // bn,),
        in_specs=[
            pl.BlockSpec((c, bn, h), lambda i: (0, i, 0)),
            pl.BlockSpec((bn, cd), lambda i: (i, 0)),
            pl.BlockSpec((1, h), lambda i: (0, 0)),
            pl.BlockSpec((h, dout), lambda i: (0, 0)),
            pl.BlockSpec((1, dout), lambda i: (0, 0)),
            pl.BlockSpec((h, dout), lambda i: (0, 0)),
            pl.BlockSpec((1, dout), lambda i: (0, 0)),
        ],
        out_specs=[
            pl.BlockSpec((bn, dout), lambda i: (i, 0)),
            pl.BlockSpec((bn, dout), lambda i: (i, 0)),
        ],
        out_shape=[
            jax.ShapeDtypeStruct((n, dout), jnp.float32),
            jax.ShapeDtypeStruct((n, dout), jnp.float32),
        ],
    )(num, den, bias.reshape(1, -1), wl, bl.reshape(1, -1), wr, br.reshape(1, -1))


def _edge_pass(xl_tab, xr_tab, att, src, dst, *, n_dst_loc, feat_split):
    ho = H // 2 if feat_split else H
    nchunks = EP // K
    per_tile = nchunks // (16 if feat_split else 32)
    nzc = n_dst_loc // K
    zc = (nzc + 15) // 16

    mesh = plsc.VectorSubcoreMesh(core_axis_name="c", subcore_axis_name="s")

    @functools.partial(
        pl.kernel,
        out_type=(
            jax.ShapeDtypeStruct((2, n_dst_loc, ho), jnp.float32),
            jax.ShapeDtypeStruct((2, n_dst_loc), jnp.float32) if feat_split
            else jax.ShapeDtypeStruct((2, 16, n_dst_loc // H, H), jnp.float32),
        ),
        mesh=mesh,
        compiler_params=pltpu.CompilerParams(needs_layout_passes=False,
                                             use_tc_tiling_on_sc=False),
        scratch_types=[
            pltpu.VMEM((H,), jnp.float32),
            pltpu.VMEM((2, K), jnp.int32),
            pltpu.VMEM((2, K), jnp.int32),
            pltpu.VMEM((2, K), jnp.int32),
            pltpu.VMEM((2, K, H), jnp.float32),
            pltpu.VMEM((2, K, H), jnp.float32),
            pltpu.VMEM((2, K, ho), jnp.float32),
            pltpu.VMEM((2, K), jnp.float32),
            pltpu.VMEM_SHARED((n_dst_loc, ho), jnp.float32),
            (pltpu.VMEM_SHARED((n_dst_loc,), jnp.float32) if feat_split
             else pltpu.VMEM((n_dst_loc // H, H), jnp.float32)),
        ] + [pltpu.SemaphoreType.DMA] * 12,
    )
    def kfn(xl_hbm, xr_hbm, att_hbm, src_hbm, dst_hbm, num_out, den_out,
            att_v, src_v, dst_v, loc_v, l_v, r_v, msg_v, ex_v,
            num_sh, den_acc,
            gl0, gl1, gr0, gr1, is0, is1, id0, id1, sn0, sn1, sd0, sd1):
        sem_gl = (gl0, gl1)
        sem_gr = (gr0, gr1)
        sem_is = (is0, is1)
        sem_id = (id0, id1)
        sem_sn = (sn0, sn1)
        sem_sd = (sd0, sd1)
        c = lax.axis_index("c")
        s = lax.axis_index("s")
        zero16 = jnp.zeros((16,), jnp.float32)
        for r in range(K):
            msg_v[0, r, 0:16] = zero16
            if not feat_split:
                msg_v[0, r, 16:32] = zero16
        for j in range(0, K, 16):
            ex_v[0, pl.ds(j, 16)] = zero16
        if not feat_split:
            def dzbody(i, _):
                for rr in range(4):
                    den_acc[i * 4 + rr, 0:16] = zero16
                    den_acc[i * 4 + rr, 16:32] = zero16
                return 0
            lax.fori_loop(0, n_dst_loc // H // 4, dzbody, 0)

        def zbody(i, _):
            ci = i * 16 + s
            @pl.when(ci < nzc)
            def _():
                row = ci * K
                pltpu.sync_copy(msg_v.at[0], num_sh.at[pl.ds(row, K)])
                if feat_split:
                    pltpu.sync_copy(ex_v.at[0], den_acc.at[pl.ds(row, K)])
            return 0
        lax.fori_loop(0, zc, zbody, 0)
        plsc.subcore_barrier()

        pltpu.sync_copy(att_hbm, att_v)
        if feat_split:
            tile_base = s * (per_tile * K)
        else:
            tile_base = (s * 2 + c) * (per_tile * K)
        is_hi = c == 1

        def gathers(chunk, b):
            pltpu.async_copy(xl_hbm.at[src_v.at[b]], l_v.at[b], sem_gl[b])
            pltpu.async_copy(xr_hbm.at[dst_v.at[b]], r_v.at[b], sem_gr[b])

        def idx_copy(chunk, b):
            eb = tile_base + chunk * K
            pltpu.async_copy(src_hbm.at[pl.ds(eb, K)], src_v.at[b], sem_is[b])
            pltpu.async_copy(dst_hbm.at[pl.ds(eb, K)], dst_v.at[b], sem_id[b])

        pltpu.sync_copy(src_hbm.at[pl.ds(tile_base, K)], src_v.at[0])
        pltpu.sync_copy(dst_hbm.at[pl.ds(tile_base, K)], dst_v.at[0])
        gathers(0, 0)
        idx_copy(1, 1)

        iota16 = lax.iota(jnp.int32, 16)

        def body(ii, _):
            for b in (0, 1):
                b1 = 1 - b
                i = 2 * ii + b
                pltpu.make_async_copy(src_hbm.at[pl.ds(0, K)], src_v.at[b1], sem_is[b1]).wait()
                pltpu.make_async_copy(dst_hbm.at[pl.ds(0, K)], dst_v.at[b1], sem_id[b1]).wait()
                gathers(i + 1, b1)
                @pl.when(ii >= 1)
                def _():
                    pltpu.make_async_copy(msg_v.at[b], num_sh.at[pl.ds(0, K)], sem_sn[b]).wait()
                if feat_split:
                    @pl.when((ii >= 1) & (c == b))
                    def _():
                        pltpu.make_async_copy(ex_v.at[b], den_acc.at[pl.ds(0, K)], sem_sd[b]).wait()
                def locbody(j, _):
                    loc_v[b, pl.ds(j * 16, 16)] = dst_v[b, pl.ds(j * 16, 16)]
                    return 0
                lax.fori_loop(0, K // 16, locbody, 0)
                pltpu.make_async_copy(xl_hbm.at[src_v.at[b]], l_v.at[b], sem_gl[b]).wait()
                pltpu.make_async_copy(xr_hbm.at[dst_v.at[b]], r_v.at[b], sem_gr[b]).wait()
                idx_copy(i + 2, b)
                att0 = att_v[pl.ds(0, 16)]
                att1 = att_v[pl.ds(16, 16)]
                def grpbody(j, _):
                    rows = iota16 + j * 16
                    sv = jnp.zeros((16,), jnp.float32)
                    for kk in range(H):
                        ksp = jnp.full((16,), kk, jnp.int32)
                        lk = plsc.load_gather(l_v.at[b], [rows, ksp])
                        rk = plsc.load_gather(r_v.at[b], [rows, ksp])
                        u = lk + rk
                        lrv = jnp.maximum(u, 0.0) + 0.2 * jnp.minimum(u, 0.0)
                        ak = att0[kk] if kk < 16 else att1[kk - 16]
                        sv = sv + ak * lrv
                    exv = jnp.exp(sv)
                    ex_v[b, pl.ds(j * 16, 16)] = exv
                    if not feat_split:
                        lv = loc_v[b, pl.ds(j * 16, 16)]
                        plsc.addupdate_scatter(den_acc, [lv >> 5, lv & 31], exv)
                    for jj in range(16):
                        e = j * 16 + jj
                        exs = exv[jj]
                        if feat_split:
                            lo = l_v[b, e, 0:16]
                            hi = l_v[b, e, 16:32]
                            msg_v[b, e, 0:16] = jnp.where(is_hi, hi, lo) * exs
                        else:
                            msg_v[b, e, 0:16] = l_v[b, e, 0:16] * exs
                            msg_v[b, e, 16:32] = l_v[b, e, 16:32] * exs
                    return 0
                lax.fori_loop(0, K // 16, grpbody, 0)
                pltpu.async_copy(msg_v.at[b], num_sh.at[loc_v.at[b]], sem_sn[b], add=True)
                if feat_split:
                    @pl.when(c == b)
                    def _():
                        pltpu.async_copy(ex_v.at[b], den_acc.at[loc_v.at[b]], sem_sd[b], add=True)
            return 0
        lax.fori_loop(0, per_tile // 2, body, 0)

        pltpu.make_async_copy(xl_hbm.at[src_v.at[0]], l_v.at[0], sem_gl[0]).wait()
        pltpu.make_async_copy(xr_hbm.at[dst_v.at[0]], r_v.at[0], sem_gr[0]).wait()
        pltpu.make_async_copy(src_hbm.at[pl.ds(0, K)], src_v.at[1], sem_is[1]).wait()
        pltpu.make_async_copy(dst_hbm.at[pl.ds(0, K)], dst_v.at[1], sem_id[1]).wait()
        for b in (0, 1):
            pltpu.make_async_copy(msg_v.at[b], num_sh.at[pl.ds(0, K)], sem_sn[b]).wait()
            if feat_split:
                @pl.when(c == b)
                def _():
                    pltpu.make_async_copy(ex_v.at[b], den_acc.at[pl.ds(0, K)], sem_sd[b]).wait()
        if not feat_split:
            pltpu.sync_copy(den_acc, den_out.at[c, s])
        plsc.subcore_barrier()

        def wbody(i, _):
            ci = i * 16 + s
            @pl.when(ci < nzc)
            def _():
                row = ci * K
                pltpu.sync_copy(num_sh.at[pl.ds(row, K)], num_out.at[c, pl.ds(row, K)])
                if feat_split:
                    pltpu.sync_copy(den_acc.at[pl.ds(row, K)], den_out.at[c, pl.ds(row, K)])
            return 0
        lax.fori_loop(0, zc, wbody, 0)

    return kfn(xl_tab, xr_tab, att, src, dst)


def _pair_pass(p_tab, q_tab, w2b, ls, ld):
    per_tile = LP // K // 32
    mesh = plsc.VectorSubcoreMesh(core_axis_name="c", subcore_axis_name="s")

    @functools.partial(
        pl.kernel,
        out_type=jax.ShapeDtypeStruct((LP,), jnp.float32),
        mesh=mesh,
        compiler_params=pltpu.CompilerParams(needs_layout_passes=False,
                                             use_tc_tiling_on_sc=False),
        scratch_types=[
            pltpu.VMEM((48,), jnp.float32),
            pltpu.VMEM((2, K), jnp.int32),
            pltpu.VMEM((2, K), jnp.int32),
            pltpu.VMEM((2, K, H), jnp.float32),
            pltpu.VMEM((2, K, H), jnp.float32),
            pltpu.VMEM((2, K), jnp.float32),
        ] + [pltpu.SemaphoreType.DMA] * 10,
    )
    def kfn(p_hbm, q_hbm, w2b_hbm, ls_hbm, ld_hbm, pred_out,
            w2b_v, ls_v, ld_v, p_v, q_v, o_v,
            gl0, gl1, gr0, gr1, is0, is1, id0, id1, so0, so1):
        sem_gl = (gl0, gl1)
        sem_gr = (gr0, gr1)
        sem_is = (is0, is1)
        sem_id = (id0, id1)
        sem_so = (so0, so1)
        c = lax.axis_index("c")
        s = lax.axis_index("s")
        tile_base = (s * 2 + c) * (per_tile * K)
        pltpu.sync_copy(w2b_hbm, w2b_v)

        def gathers(chunk, b):
            pltpu.async_copy(p_hbm.at[ls_v.at[b]], p_v.at[b], sem_gl[b])
            pltpu.async_copy(q_hbm.at[ld_v.at[b]], q_v.at[b], sem_gr[b])

        def idx_copy(chunk, b):
            eb = tile_base + chunk * K
            pltpu.async_copy(ls_hbm.at[pl.ds(eb, K)], ls_v.at[b], sem_is[b])
            pltpu.async_copy(ld_hbm.at[pl.ds(eb, K)], ld_v.at[b], sem_id[b])

        pltpu.sync_copy(ls_hbm.at[pl.ds(tile_base, K)], ls_v.at[0])
        pltpu.sync_copy(ld_hbm.at[pl.ds(tile_base, K)], ld_v.at[0])
        gathers(0, 0)
        idx_copy(1, 1)
        iota16 = lax.iota(jnp.int32, 16)

        def body(ii, _):
            for b in (0, 1):
                b1 = 1 - b
                i = 2 * ii + b
                pltpu.make_async_copy(ls_hbm.at[pl.ds(0, K)], ls_v.at[b1], sem_is[b1]).wait()
                pltpu.make_async_copy(ld_hbm.at[pl.ds(0, K)], ld_v.at[b1], sem_id[b1]).wait()
                gathers(i + 1, b1)
                @pl.when(ii >= 1)
                def _():
                    pltpu.make_async_copy(o_v.at[b], pred_out.at[pl.ds(0, K)], sem_so[b]).wait()
                pltpu.make_async_copy(p_hbm.at[ls_v.at[b]], p_v.at[b], sem_gl[b]).wait()
                pltpu.make_async_copy(q_hbm.at[ld_v.at[b]], q_v.at[b], sem_gr[b]).wait()
                idx_copy(i + 2, b)
                w20 = w2b_v[pl.ds(0, 16)]
                w21 = w2b_v[pl.ds(16, 16)]
                b2 = w2b_v[pl.ds(32, 16)][0]
                def grpbody(j, _):
                    rows = iota16 + j * 16
                    sv = jnp.full((16,), b2, jnp.float32)
                    for kk in range(H):
                        ksp = jnp.full((16,), kk, jnp.int32)
                        pk = plsc.load_gather(p_v.at[b], [rows, ksp])
                        qk = plsc.load_gather(q_v.at[b], [rows, ksp])
                        hk = jnp.maximum(pk + qk, 0.0)
                        wk = w20[kk] if kk < 16 else w21[kk - 16]
                        sv = sv + wk * hk
                    o_v[b, pl.ds(j * 16, 16)] = sv
                    return 0
                lax.fori_loop(0, K // 16, grpbody, 0)
                eb = tile_base + i * K
                pltpu.async_copy(o_v.at[b], pred_out.at[pl.ds(eb, K)], sem_so[b])
            return 0
        lax.fori_loop(0, per_tile // 2, body, 0)

        pltpu.make_async_copy(p_hbm.at[ls_v.at[0]], p_v.at[0], sem_gl[0]).wait()
        pltpu.make_async_copy(q_hbm.at[ld_v.at[0]], q_v.at[0], sem_gr[0]).wait()
        pltpu.make_async_copy(ls_hbm.at[pl.ds(0, K)], ls_v.at[1], sem_is[1]).wait()
        pltpu.make_async_copy(ld_hbm.at[pl.ds(0, K)], ld_v.at[1], sem_id[1]).wait()
        for b in (0, 1):
            pltpu.make_async_copy(o_v.at[b], pred_out.at[pl.ds(0, K)], sem_so[b]).wait()

    return kfn(p_tab, q_tab, w2b, ls, ld)


def _pad_rows(x, n):
    return jnp.pad(x, ((0, n - x.shape[0]), (0, 0)))


def _pad_idx(x, n, fill):
    return jnp.concatenate([x.astype(jnp.int32),
                            jnp.full((n - x.shape[0],), fill, jnp.int32)])


def _user_nd(num, den):
    nu_p = 100352
    n = jnp.concatenate([num[0], num[1]], axis=-1)
    n = _pad_rows(n, nu_p)[None]
    d = jnp.pad(den, ((0, 0), (0, nu_p - NLOC_U)), constant_values=0.5).T
    return n, d


def kernel(x_user, x_item, params, src_u2i, dst_u2i, src_i2u, dst_i2u,
           label_src, label_dst):
    p = params
    c1u, c1i = p['c1_u2i'], p['c1_i2u']
    c2u, c2i = p['c2_u2i'], p['c2_i2u']

    xu = _pad_rows(x_user, 100352)
    xi = _pad_rows(x_item, 10240)

    xl_u, xr_u = _mm2(xu, c1u['Wl'], c1u['bl'], c1i['Wr'], c1i['br'])
    xl_i, xr_i = _mm2(xi, c1i['Wl'], c1i['bl'], c1u['Wr'], c1u['br'])

    su2i = _pad_idx(src_u2i, EP2, 0)
    du2i = _pad_idx(dst_u2i, EP2, NI)
    si2u = _pad_idx(src_i2u, EP2, 0)
    di2u = _pad_idx(dst_i2u, EP2, NU)

    num_u, den_u = _edge_pass(xl_i, xr_u, c1i['att'], si2u, di2u,
                              n_dst_loc=NLOC_U, feat_split=True)
    num_i, den_i = _edge_pass(xl_u, xr_i, c1u['att'], su2i, du2i,
                              n_dst_loc=NLOC_I, feat_split=False)

    nu_, du_ = _user_nd(num_u, den_u)
    xl_u2, xr_u2 = _cmm(nu_, du_, c1i['bias'],
                        c2u['Wl'], c2u['bl'], c2i['Wr'], c2i['br'], relu=True)
    xl_i2, xr_i2 = _cmm(num_i, den_i.reshape(32, NLOC_I).T, c1u['bias'],
                        c2i['Wl'], c2i['bl'], c2u['Wr'], c2u['br'], relu=True)

    num_u2, den_u2 = _edge_pass(xl_i2, xr_u2, c2i['att'], si2u, di2u,
                                n_dst_loc=NLOC_U, feat_split=True)
    num_i2, den_i2 = _edge_pass(xl_u2, xr_i2, c2u['att'], su2i, du2i,
                                n_dst_loc=NLOC_I, feat_split=False)

    w1a = p['dec_W1'][:H, :]
    w1b = p['dec_W1'][H:, :]
    zero_h = jnp.zeros((H,), jnp.float32)
    nu2_, du2_ = _user_nd(num_u2, den_u2)
    p_tab, _ = _cmm(nu2_, du2_, c2i['bias'], w1a, zero_h, w1a, zero_h, relu=False)
    q_tab, _ = _cmm(num_i2, den_i2.reshape(32, NLOC_I).T, c2u['bias'],
                    w1b, p['dec_b1'], w1b, p['dec_b1'], relu=False)

    w2b = jnp.concatenate([p['dec_W2'][:, 0], p['dec_b2'],
                           jnp.zeros((15,), jnp.float32)])
    ls = _pad_idx(label_src, LP2, 0)
    ld = _pad_idx(label_dst, LP2, 0)
    pred = _pair_pass(p_tab, q_tab, w2b, ls, ld)[:LBL]
    mask = jnp.ones((LBL,), dtype=bool)
    return pred, mask


# speedup vs baseline: 1.0055x; 1.0055x over previous
import functools

import jax
import jax.numpy as jnp
from jax import lax
from jax.experimental import pallas as pl
from jax.experimental.pallas import tpu as pltpu
from jax.experimental.pallas import tpu_sc as plsc

NU = 100000
NI = 10000
EDG = 1600000
DIN = 128
H = 32
LBL = 400000

K = 128
EP = 1605632
EP2 = EP + 2 * K
LP = 401408
LP2 = LP + 2 * K
NLOC_U = 100096
NLOC_I = 12288
HALF_U = 50000


def _mm2_body(x_ref, wl_ref, bl_ref, wr_ref, br_ref, ol_ref, or_ref):
    x = x_ref[...]
    ol_ref[...] = jnp.dot(x, wl_ref[...], preferred_element_type=jnp.float32) + bl_ref[...]
    or_ref[...] = jnp.dot(x, wr_ref[...], preferred_element_type=jnp.float32) + br_ref[...]


def _mm2(x, wl, bl, wr, br, bn=2048):
    n, din = x.shape
    dout = wl.shape[1]
    return pl.pallas_call(
        _mm2_body,
        grid=(n // bn,),
        in_specs=[
            pl.BlockSpec((bn, din), lambda i: (i, 0)),
            pl.BlockSpec((din, dout), lambda i: (0, 0)),
            pl.BlockSpec((1, dout), lambda i: (0, 0)),
            pl.BlockSpec((din, dout), lambda i: (0, 0)),
            pl.BlockSpec((1, dout), lambda i: (0, 0)),
        ],
        out_specs=[
            pl.BlockSpec((bn, dout), lambda i: (i, 0)),
            pl.BlockSpec((bn, dout), lambda i: (i, 0)),
        ],
        out_shape=[
            jax.ShapeDtypeStruct((n, dout), jnp.float32),
            jax.ShapeDtypeStruct((n, dout), jnp.float32),
        ],
    )(x, wl, bl.reshape(1, -1), wr, br.reshape(1, -1))


def _cmm_body(num_ref, den_ref, bias_ref, wl_ref, bl_ref, wr_ref, br_ref,
              ol_ref, or_ref, *, relu):
    num = jnp.sum(num_ref[...], axis=0)
    den = jnp.sum(den_ref[...], axis=1, keepdims=True) + 1e-30
    z = num / den + bias_ref[...]
    if relu:
        z = jnp.maximum(z, 0.0)
    ol_ref[...] = jnp.dot(z, wl_ref[...], preferred_element_type=jnp.float32) + bl_ref[...]
    or_ref[...] = jnp.dot(z, wr_ref[...], preferred_element_type=jnp.float32) + br_ref[...]


def _cmm(num, den, bias, wl, bl, wr, br, relu, bn=2048):
    c, n, h = num.shape
    cd = den.shape[1]
    dout = wl.shape[1]
    return pl.pallas_call(
        functools.partial(_cmm_body, relu=relu),
        grid=(n // bn,),
        in_specs=[
            pl.BlockSpec((c, bn, h), lambda i: (0, i, 0)),
            pl.BlockSpec((bn, cd), lambda i: (i, 0)),
            pl.BlockSpec((1, h), lambda i: (0, 0)),
            pl.BlockSpec((h, dout), lambda i: (0, 0)),
            pl.BlockSpec((1, dout), lambda i: (0, 0)),
            pl.BlockSpec((h, dout), lambda i: (0, 0)),
            pl.BlockSpec((1, dout), lambda i: (0, 0)),
        ],
        out_specs=[
            pl.BlockSpec((bn, dout), lambda i: (i, 0)),
            pl.BlockSpec((bn, dout), lambda i: (i, 0)),
        ],
        out_shape=[
            jax.ShapeDtypeStruct((n, dout), jnp.float32),
            jax.ShapeDtypeStruct((n, dout), jnp.float32),
        ],
    )(num, den, bias.reshape(1, -1), wl, bl.reshape(1, -1), wr, br.reshape(1, -1))


def _edge_pass(xl_tab, xr_tab, att, src, dst, *, n_dst_loc, feat_split):
    ho = H // 2 if feat_split else H
    nchunks = EP // K
    per_tile = nchunks // (16 if feat_split else 32)
    nzc = n_dst_loc // K
    zc = (nzc + 15) // 16

    mesh = plsc.VectorSubcoreMesh(core_axis_name="c", subcore_axis_name="s")

    @functools.partial(
        pl.kernel,
        out_type=(
            jax.ShapeDtypeStruct((2, n_dst_loc, ho), jnp.float32),
            jax.ShapeDtypeStruct((2, n_dst_loc), jnp.float32) if feat_split
            else jax.ShapeDtypeStruct((2, 16, n_dst_loc // H, H), jnp.float32),
        ),
        mesh=mesh,
        compiler_params=pltpu.CompilerParams(needs_layout_passes=False,
                                             use_tc_tiling_on_sc=False),
        scratch_types=[
            pltpu.VMEM((H,), jnp.float32),
            pltpu.VMEM((2, K), jnp.int32),
            pltpu.VMEM((2, K), jnp.int32),
            pltpu.VMEM((2, K), jnp.int32),
            pltpu.VMEM((2, K, H), jnp.float32),
            pltpu.VMEM((2, K, H), jnp.float32),
            pltpu.VMEM((2, K, ho), jnp.float32),
            pltpu.VMEM((2, K), jnp.float32),
            pltpu.VMEM_SHARED((n_dst_loc, ho), jnp.float32),
            (pltpu.VMEM_SHARED((n_dst_loc,), jnp.float32) if feat_split
             else pltpu.VMEM((n_dst_loc // H, H), jnp.float32)),
        ] + [pltpu.SemaphoreType.DMA] * 12,
    )
    def kfn(xl_hbm, xr_hbm, att_hbm, src_hbm, dst_hbm, num_out, den_out,
            att_v, src_v, dst_v, loc_v, l_v, r_v, msg_v, ex_v,
            num_sh, den_acc,
            gl0, gl1, gr0, gr1, is0, is1, id0, id1, sn0, sn1, sd0, sd1):
        sem_gl = (gl0, gl1)
        sem_gr = (gr0, gr1)
        sem_is = (is0, is1)
        sem_id = (id0, id1)
        sem_sn = (sn0, sn1)
        sem_sd = (sd0, sd1)
        c = lax.axis_index("c")
        s = lax.axis_index("s")
        zero16 = jnp.zeros((16,), jnp.float32)
        for r in range(K):
            msg_v[0, r, 0:16] = zero16
            if not feat_split:
                msg_v[0, r, 16:32] = zero16
        for j in range(0, K, 16):
            ex_v[0, pl.ds(j, 16)] = zero16
        if not feat_split:
            def dzbody(i, _):
                for rr in range(4):
                    den_acc[i * 4 + rr, 0:16] = zero16
                    den_acc[i * 4 + rr, 16:32] = zero16
                return 0
            lax.fori_loop(0, n_dst_loc // H // 4, dzbody, 0)

        def zbody(i, _):
            ci = i * 16 + s
            @pl.when(ci < nzc)
            def _():
                row = ci * K
                pltpu.sync_copy(msg_v.at[0], num_sh.at[pl.ds(row, K)])
                if feat_split:
                    pltpu.sync_copy(ex_v.at[0], den_acc.at[pl.ds(row, K)])
            return 0
        lax.fori_loop(0, zc, zbody, 0)
        plsc.subcore_barrier()

        pltpu.sync_copy(att_hbm, att_v)
        if feat_split:
            tile_base = s * (per_tile * K)
        else:
            tile_base = (s * 2 + c) * (per_tile * K)
        is_hi = c == 1

        def gathers(chunk, b):
            pltpu.async_copy(xl_hbm.at[src_v.at[b]], l_v.at[b], sem_gl[b])
            pltpu.async_copy(xr_hbm.at[dst_v.at[b]], r_v.at[b], sem_gr[b])

        def idx_copy(chunk, b):
            eb = tile_base + chunk * K
            pltpu.async_copy(src_hbm.at[pl.ds(eb, K)], src_v.at[b], sem_is[b])
            pltpu.async_copy(dst_hbm.at[pl.ds(eb, K)], dst_v.at[b], sem_id[b])

        pltpu.sync_copy(src_hbm.at[pl.ds(tile_base, K)], src_v.at[0])
        pltpu.sync_copy(dst_hbm.at[pl.ds(tile_base, K)], dst_v.at[0])
        gathers(0, 0)
        idx_copy(1, 1)

        iota16 = lax.iota(jnp.int32, 16)

        def body(ii, _):
            for b in (0, 1):
                b1 = 1 - b
                i = 2 * ii + b
                pltpu.make_async_copy(src_hbm.at[pl.ds(0, K)], src_v.at[b1], sem_is[b1]).wait()
                pltpu.make_async_copy(dst_hbm.at[pl.ds(0, K)], dst_v.at[b1], sem_id[b1]).wait()
                gathers(i + 1, b1)
                @pl.when(ii >= 1)
                def _():
                    pltpu.make_async_copy(msg_v.at[b], num_sh.at[pl.ds(0, K)], sem_sn[b]).wait()
                if feat_split:
                    @pl.when((ii >= 1) & (c == b))
                    def _():
                        pltpu.make_async_copy(ex_v.at[b], den_acc.at[pl.ds(0, K)], sem_sd[b]).wait()
                def locbody(j, _):
                    loc_v[b, pl.ds(j * 16, 16)] = dst_v[b, pl.ds(j * 16, 16)]
                    return 0
                lax.fori_loop(0, K // 16, locbody, 0)
                pltpu.make_async_copy(xl_hbm.at[src_v.at[b]], l_v.at[b], sem_gl[b]).wait()
                pltpu.make_async_copy(xr_hbm.at[dst_v.at[b]], r_v.at[b], sem_gr[b]).wait()
                idx_copy(i + 2, b)
                att0 = att_v[pl.ds(0, 16)]
                att1 = att_v[pl.ds(16, 16)]
                def grpbody(j, _):
                    rows = iota16 + j * 16
                    acc = [jnp.zeros((16,), jnp.float32) for _ in range(4)]
                    for kk in range(H):
                        ksp = jnp.full((16,), kk, jnp.int32)
                        lk = plsc.load_gather(l_v.at[b], [rows, ksp])
                        rk = plsc.load_gather(r_v.at[b], [rows, ksp])
                        u = lk + rk
                        lrv = jnp.maximum(u, 0.0) + 0.2 * jnp.minimum(u, 0.0)
                        ak = att0[kk] if kk < 16 else att1[kk - 16]
                        acc[kk % 4] = acc[kk % 4] + ak * lrv
                    sv = (acc[0] + acc[1]) + (acc[2] + acc[3])
                    exv = jnp.exp(sv)
                    ex_v[b, pl.ds(j * 16, 16)] = exv
                    if not feat_split:
                        lv = loc_v[b, pl.ds(j * 16, 16)]
                        plsc.addupdate_scatter(den_acc, [lv >> 5, lv & 31], exv)
                    for jj in range(16):
                        e = j * 16 + jj
                        exs = exv[jj]
                        if feat_split:
                            lo = l_v[b, e, 0:16]
                            hi = l_v[b, e, 16:32]
                            msg_v[b, e, 0:16] = jnp.where(is_hi, hi, lo) * exs
                        else:
                            msg_v[b, e, 0:16] = l_v[b, e, 0:16] * exs
                            msg_v[b, e, 16:32] = l_v[b, e, 16:32] * exs
                    return 0
                lax.fori_loop(0, K // 16, grpbody, 0)
                pltpu.async_copy(msg_v.at[b], num_sh.at[loc_v.at[b]], sem_sn[b], add=True)
                if feat_split:
                    @pl.when(c == b)
                    def _():
                        pltpu.async_copy(ex_v.at[b], den_acc.at[loc_v.at[b]], sem_sd[b], add=True)
            return 0
        lax.fori_loop(0, per_tile // 2, body, 0)

        pltpu.make_async_copy(xl_hbm.at[src_v.at[0]], l_v.at[0], sem_gl[0]).wait()
        pltpu.make_async_copy(xr_hbm.at[dst_v.at[0]], r_v.at[0], sem_gr[0]).wait()
        pltpu.make_async_copy(src_hbm.at[pl.ds(0, K)], src_v.at[1], sem_is[1]).wait()
        pltpu.make_async_copy(dst_hbm.at[pl.ds(0, K)], dst_v.at[1], sem_id[1]).wait()
        for b in (0, 1):
            pltpu.make_async_copy(msg_v.at[b], num_sh.at[pl.ds(0, K)], sem_sn[b]).wait()
            if feat_split:
                @pl.when(c == b)
                def _():
                    pltpu.make_async_copy(ex_v.at[b], den_acc.at[pl.ds(0, K)], sem_sd[b]).wait()
        if not feat_split:
            pltpu.sync_copy(den_acc, den_out.at[c, s])
        plsc.subcore_barrier()

        def wbody(i, _):
            ci = i * 16 + s
            @pl.when(ci < nzc)
            def _():
                row = ci * K
                pltpu.sync_copy(num_sh.at[pl.ds(row, K)], num_out.at[c, pl.ds(row, K)])
                if feat_split:
                    pltpu.sync_copy(den_acc.at[pl.ds(row, K)], den_out.at[c, pl.ds(row, K)])
            return 0
        lax.fori_loop(0, zc, wbody, 0)

    return kfn(xl_tab, xr_tab, att, src, dst)


def _pair_pass(p_tab, q_tab, w2b, ls, ld):
    per_tile = LP // K // 32
    mesh = plsc.VectorSubcoreMesh(core_axis_name="c", subcore_axis_name="s")

    @functools.partial(
        pl.kernel,
        out_type=jax.ShapeDtypeStruct((LP,), jnp.float32),
        mesh=mesh,
        compiler_params=pltpu.CompilerParams(needs_layout_passes=False,
                                             use_tc_tiling_on_sc=False),
        scratch_types=[
            pltpu.VMEM((48,), jnp.float32),
            pltpu.VMEM((2, K), jnp.int32),
            pltpu.VMEM((2, K), jnp.int32),
            pltpu.VMEM((2, K, H), jnp.float32),
            pltpu.VMEM((2, K, H), jnp.float32),
            pltpu.VMEM((2, K), jnp.float32),
        ] + [pltpu.SemaphoreType.DMA] * 10,
    )
    def kfn(p_hbm, q_hbm, w2b_hbm, ls_hbm, ld_hbm, pred_out,
            w2b_v, ls_v, ld_v, p_v, q_v, o_v,
            gl0, gl1, gr0, gr1, is0, is1, id0, id1, so0, so1):
        sem_gl = (gl0, gl1)
        sem_gr = (gr0, gr1)
        sem_is = (is0, is1)
        sem_id = (id0, id1)
        sem_so = (so0, so1)
        c = lax.axis_index("c")
        s = lax.axis_index("s")
        tile_base = (s * 2 + c) * (per_tile * K)
        pltpu.sync_copy(w2b_hbm, w2b_v)

        def gathers(chunk, b):
            pltpu.async_copy(p_hbm.at[ls_v.at[b]], p_v.at[b], sem_gl[b])
            pltpu.async_copy(q_hbm.at[ld_v.at[b]], q_v.at[b], sem_gr[b])

        def idx_copy(chunk, b):
            eb = tile_base + chunk * K
            pltpu.async_copy(ls_hbm.at[pl.ds(eb, K)], ls_v.at[b], sem_is[b])
            pltpu.async_copy(ld_hbm.at[pl.ds(eb, K)], ld_v.at[b], sem_id[b])

        pltpu.sync_copy(ls_hbm.at[pl.ds(tile_base, K)], ls_v.at[0])
        pltpu.sync_copy(ld_hbm.at[pl.ds(tile_base, K)], ld_v.at[0])
        gathers(0, 0)
        idx_copy(1, 1)
        iota16 = lax.iota(jnp.int32, 16)

        def body(ii, _):
            for b in (0, 1):
                b1 = 1 - b
                i = 2 * ii + b
                pltpu.make_async_copy(ls_hbm.at[pl.ds(0, K)], ls_v.at[b1], sem_is[b1]).wait()
                pltpu.make_async_copy(ld_hbm.at[pl.ds(0, K)], ld_v.at[b1], sem_id[b1]).wait()
                gathers(i + 1, b1)
                @pl.when(ii >= 1)
                def _():
                    pltpu.make_async_copy(o_v.at[b], pred_out.at[pl.ds(0, K)], sem_so[b]).wait()
                pltpu.make_async_copy(p_hbm.at[ls_v.at[b]], p_v.at[b], sem_gl[b]).wait()
                pltpu.make_async_copy(q_hbm.at[ld_v.at[b]], q_v.at[b], sem_gr[b]).wait()
                idx_copy(i + 2, b)
                w20 = w2b_v[pl.ds(0, 16)]
                w21 = w2b_v[pl.ds(16, 16)]
                b2 = w2b_v[pl.ds(32, 16)][0]
                def grpbody(j, _):
                    rows = iota16 + j * 16
                    acc = [jnp.zeros((16,), jnp.float32) for _ in range(4)]
                    for kk in range(H):
                        ksp = jnp.full((16,), kk, jnp.int32)
                        pk = plsc.load_gather(p_v.at[b], [rows, ksp])
                        qk = plsc.load_gather(q_v.at[b], [rows, ksp])
                        hk = jnp.maximum(pk + qk, 0.0)
                        wk = w20[kk] if kk < 16 else w21[kk - 16]
                        acc[kk % 4] = acc[kk % 4] + wk * hk
                    o_v[b, pl.ds(j * 16, 16)] = ((acc[0] + acc[1]) + (acc[2] + acc[3])) + b2
                    return 0
                lax.fori_loop(0, K // 16, grpbody, 0)
                eb = tile_base + i * K
                pltpu.async_copy(o_v.at[b], pred_out.at[pl.ds(eb, K)], sem_so[b])
            return 0
        lax.fori_loop(0, per_tile // 2, body, 0)

        pltpu.make_async_copy(p_hbm.at[ls_v.at[0]], p_v.at[0], sem_gl[0]).wait()
        pltpu.make_async_copy(q_hbm.at[ld_v.at[0]], q_v.at[0], sem_gr[0]).wait()
        pltpu.make_async_copy(ls_hbm.at[pl.ds(0, K)], ls_v.at[1], sem_is[1]).wait()
        pltpu.make_async_copy(ld_hbm.at[pl.ds(0, K)], ld_v.at[1], sem_id[1]).wait()
        for b in (0, 1):
            pltpu.make_async_copy(o_v.at[b], pred_out.at[pl.ds(0, K)], sem_so[b]).wait()

    return kfn(p_tab, q_tab, w2b, ls, ld)


def _pad_rows(x, n):
    return jnp.pad(x, ((0, n - x.shape[0]), (0, 0)))


def _pad_idx(x, n, fill):
    return jnp.concatenate([x.astype(jnp.int32),
                            jnp.full((n - x.shape[0],), fill, jnp.int32)])


def _user_nd(num, den):
    nu_p = 100352
    n = jnp.concatenate([num[0], num[1]], axis=-1)
    n = _pad_rows(n, nu_p)[None]
    d = jnp.pad(den, ((0, 0), (0, nu_p - NLOC_U)), constant_values=0.5).T
    return n, d


def kernel(x_user, x_item, params, src_u2i, dst_u2i, src_i2u, dst_i2u,
           label_src, label_dst):
    p = params
    c1u, c1i = p['c1_u2i'], p['c1_i2u']
    c2u, c2i = p['c2_u2i'], p['c2_i2u']

    xu = _pad_rows(x_user, 100352)
    xi = _pad_rows(x_item, 10240)

    xl_u, xr_u = _mm2(xu, c1u['Wl'], c1u['bl'], c1i['Wr'], c1i['br'])
    xl_i, xr_i = _mm2(xi, c1i['Wl'], c1i['bl'], c1u['Wr'], c1u['br'])

    su2i = _pad_idx(src_u2i, EP2, 0)
    du2i = _pad_idx(dst_u2i, EP2, NI)
    si2u = _pad_idx(src_i2u, EP2, 0)
    di2u = _pad_idx(dst_i2u, EP2, NU)

    num_u, den_u = _edge_pass(xl_i, xr_u, c1i['att'], si2u, di2u,
                              n_dst_loc=NLOC_U, feat_split=True)
    num_i, den_i = _edge_pass(xl_u, xr_i, c1u['att'], su2i, du2i,
                              n_dst_loc=NLOC_I, feat_split=False)

    nu_, du_ = _user_nd(num_u, den_u)
    xl_u2, xr_u2 = _cmm(nu_, du_, c1i['bias'],
                        c2u['Wl'], c2u['bl'], c2i['Wr'], c2i['br'], relu=True)
    xl_i2, xr_i2 = _cmm(num_i, den_i.reshape(32, NLOC_I).T, c1u['bias'],
                        c2i['Wl'], c2i['bl'], c2u['Wr'], c2u['br'], relu=True)

    num_u2, den_u2 = _edge_pass(xl_i2, xr_u2, c2i['att'], si2u, di2u,
                                n_dst_loc=NLOC_U, feat_split=True)
    num_i2, den_i2 = _edge_pass(xl_u2, xr_i2, c2u['att'], su2i, du2i,
                                n_dst_loc=NLOC_I, feat_split=False)

    w1a = p['dec_W1'][:H, :]
    w1b = p['dec_W1'][H:, :]
    zero_h = jnp.zeros((H,), jnp.float32)
    nu2_, du2_ = _user_nd(num_u2, den_u2)
    p_tab, _ = _cmm(nu2_, du2_, c2i['bias'], w1a, zero_h, w1a, zero_h, relu=False)
    q_tab, _ = _cmm(num_i2, den_i2.reshape(32, NLOC_I).T, c2u['bias'],
                    w1b, p['dec_b1'], w1b, p['dec_b1'], relu=False)

    w2b = jnp.concatenate([p['dec_W2'][:, 0], p['dec_b2'],
                           jnp.zeros((15,), jnp.float32)])
    ls = _pad_idx(label_src, LP2, 0)
    ld = _pad_idx(label_dst, LP2, 0)
    pred = _pair_pass(p_tab, q_tab, w2b, ls, ld)[:LBL]
    mask = jnp.ones((LBL,), dtype=bool)
    return pred, mask
